# baseline (speedup 1.0000x reference)
import functools

import jax
import jax.numpy as jnp
from jax import lax
from jax.experimental import pallas as pl
from jax.experimental.pallas import tpu as pltpu
from jax.experimental.pallas import tpu_sc as plsc

_NC = 80
_NA = 3
_CH = _NA * (5 + _NC)
_IMG = 640.0
_L_NOOBJ = 1.0
_L_OBJ = 1.0
_L_CLS = 0.5
_L_BOX = 5.0
_F_ALPHA = 0.25
_F_GAMMA = 2.0

_B = 8
_NB = 32
_NBOX = _B * _NB
_SCALES = ((80, 80), (40, 40), (20, 20))


def _bce(x, t):
    return jnp.maximum(x, 0.0) - x * t + jnp.log1p(jnp.exp(-jnp.abs(x)))


def _box_geom(x1, y1, x2, y2, s):
    h, w = _SCALES[s]
    bw = jnp.clip((x2 - x1) / _IMG, 1e-6, 1.0)
    bh = jnp.clip((y2 - y1) / _IMG, 1e-6, 1.0)
    cx = jnp.clip((x1 + x2) * 0.5 / _IMG, 0.0, 1.0 - 1e-6)
    cy = jnp.clip((y1 + y2) * 0.5 / _IMG, 0.0, 1.0 - 1e-6)
    gi = jnp.clip(jnp.floor(cx * w).astype(jnp.int32), 0, w - 1)
    gj = jnp.clip(jnp.floor(cy * h).astype(jnp.int32), 0, h - 1)
    return cx, cy, bw, bh, gi, gj


def _scale_sel(x1, y1, x2, y2, labels):
    bw = jnp.clip((x2 - x1) / _IMG, 1e-6, 1.0)
    bh = jnp.clip((y2 - y1) / _IMG, 1e-6, 1.0)
    ms = jnp.maximum(bw, bh)
    scale_idx = jnp.clip(
        jnp.where(ms < 0.15, 0, jnp.where(ms < 0.45, 1, _NA - 1)), 0, _NA - 1)
    valid = (labels >= 0) & (labels < _NC)
    return scale_idx, valid


def _prep_body(boxes_ref, idx_ref):
    boxes = boxes_ref[...]
    x1 = boxes[:, 0:1]
    y1 = boxes[:, 1:2]
    x2 = boxes[:, 2:3]
    y2 = boxes[:, 3:4]
    bidx = lax.broadcasted_iota(jnp.int32, (_NBOX, 1), 0) // _NB
    c_iota = lax.broadcasted_iota(jnp.int32, (_NBOX, 256), 1)
    for s, (h, w) in enumerate(_SCALES):
        hw = h * w
        n_elems = _B * _CH * hw
        _, _, _, _, gi, gj = _box_geom(x1, y1, x2, y2, s)
        off = gj * w + gi
        base = bidx * (_CH * hw) + off
        idx_s = jnp.minimum(base + c_iota * hw, n_elems - 1)
        idx_ref[s * _NBOX:(s + 1) * _NBOX, :] = idx_s


def _prep(boxes2):
    return pl.pallas_call(
        _prep_body,
        out_shape=jax.ShapeDtypeStruct((3 * _NBOX, 256), jnp.int32),
    )(boxes2)


def _sc_gather_body(t3, t4, t5, idx_hbm, out_hbm, idx_v, ostage, sem):
    wid = lax.axis_index("s") * 2 + lax.axis_index("c")
    for s in range(3):
        pltpu.sync_copy(idx_hbm.at[pl.ds(s * 512 + wid * 16, 16)],
                        idx_v.at[pl.ds(s * 16, 16)])
    for s, table in enumerate((t3, t4, t5)):
        for j in range(8):
            c1 = pltpu.async_copy(table.at[idx_v.at[s * 16 + j * 2]],
                                  ostage.at[pl.ds(0, 128)], sem)
            c2 = pltpu.async_copy(table.at[idx_v.at[s * 16 + j * 2 + 1]],
                                  ostage.at[pl.ds(128, 128)], sem)
            c1.wait()
            c2.wait()
            pltpu.sync_copy(
                ostage,
                out_hbm.at[pl.ds((s * _NBOX + wid * 8 + j) * 256, 256)])


def _sc_gather(t3, t4, t5, idx2):
    mesh = plsc.VectorSubcoreMesh(core_axis_name="c", subcore_axis_name="s")
    f = functools.partial(
        pl.kernel,
        mesh=mesh,
        out_type=jax.ShapeDtypeStruct((3 * _NBOX * 256,), jnp.float32),
        scratch_types=[
            pltpu.VMEM((48, 128), jnp.int32),
            pltpu.VMEM((256,), jnp.float32),
            pltpu.SemaphoreType.DMA,
        ],
    )(_sc_gather_body)
    return f(t3, t4, t5, idx2)


def _obj_partials(pred_ref, boxes, labels, s):
    h, w = _SCALES[s]
    hw = h * w
    x1 = boxes[:, 0:1]
    y1 = boxes[:, 1:2]
    x2 = boxes[:, 2:3]
    y2 = boxes[:, 3:4]
    scale_idx, valid = _scale_sel(x1, y1, x2, y2, labels)
    sel = ((scale_idx == s) & valid).astype(jnp.float32)
    _, _, _, _, gi, gj = _box_geom(x1, y1, x2, y2, s)
    cell = gj * w + gi

    iota_hw = lax.broadcasted_iota(jnp.int32, (_NB, hw), 1)
    onehot = (iota_hw == cell).astype(jnp.float32) * sel
    match = (jnp.sum(onehot, axis=0, keepdims=True) > 0.5).astype(jnp.float32)

    pos_e = jnp.float32(0.0)
    neg_e = jnp.float32(0.0)
    for a in range(_NA):
        x_obj = pred_ref[0, a, 4:5, :]
        t = match
        bce_o = _bce(x_obj, t)
        p = jax.nn.sigmoid(x_obj)
        p_t = p * t + (1.0 - p) * (1.0 - t)
        alpha_t = _F_ALPHA * t + (1.0 - _F_ALPHA) * (1.0 - t)
        one_m = 1.0 - p_t
        elem = bce_o * (alpha_t * one_m * one_m)
        pos_e = pos_e + jnp.sum(elem * match)
        neg_e = neg_e + jnp.sum(elem * (1.0 - match))
    cells = _NA * jnp.sum(match)
    return pos_e, neg_e, cells


def _sparse_terms(g, boxes_ref, labels_ref, boxest_ref, labelst_ref):
    boxes = boxes_ref[...]
    labels = labels_ref[...]
    boxest = boxest_ref[...]
    labelst = labelst_ref[...]

    def per_box(x1, y1, x2, y2, lab, bidx):
        scale_idx, valid = _scale_sel(x1, y1, x2, y2, lab)
        selv = (valid).astype(jnp.float32)
        cells = []
        for s, (h, w) in enumerate(_SCALES):
            _, _, _, _, gi, gj = _box_geom(x1, y1, x2, y2, s)
            cells.append(gj * w + gi)
        cell_own = jnp.where(scale_idx == 0, cells[0],
                             jnp.where(scale_idx == 1, cells[1], cells[2]))
        key = bidx * (3 * 6400) + scale_idx * 6400 + cell_own
        labc = jnp.clip(lab, 0, _NC - 1)
        keylab = key * _NC + labc
        return scale_idx, selv, key, keylab, labc

    x1 = boxes[:, 0:1]
    y1 = boxes[:, 1:2]
    x2 = boxes[:, 2:3]
    y2 = boxes[:, 3:4]
    bidx = lax.broadcasted_iota(jnp.int32, (_NBOX, 1), 0) // _NB
    scale_idx, selv, key, keylab, labc = per_box(x1, y1, x2, y2, labels, bidx)

    bidx_t = lax.broadcasted_iota(jnp.int32, (1, _NBOX), 1) // _NB
    _, selv_t, key_t, keylab_t, _ = per_box(
        boxest[0:1, :], boxest[1:2, :], boxest[2:3, :], boxest[3:4, :],
        labelst, bidx_t)

    i_iota = lax.broadcasted_iota(jnp.int32, (_NBOX, _NBOX), 0)
    j_iota = lax.broadcasted_iota(jnp.int32, (_NBOX, _NBOX), 1)
    tri = (j_iota < i_iota).astype(jnp.float32)
    eq_cell = ((key == key_t) & True).astype(jnp.float32) * selv_t
    dup_cell = jnp.max(tri * eq_cell, axis=1, keepdims=True)
    rep_cell = selv * (1.0 - dup_cell)
    eq_cl = (keylab == keylab_t).astype(jnp.float32) * selv_t
    dup_cl = jnp.max(tri * eq_cl, axis=1, keepdims=True)
    rep_cl = selv * (1.0 - dup_cl)

    w0 = (scale_idx == 0).astype(jnp.float32)
    w1 = (scale_idx == 1).astype(jnp.float32)
    w2 = (scale_idx == 2).astype(jnp.float32)
    gsel = (w0 * g[0:_NBOX, :] + w1 * g[_NBOX:2 * _NBOX, :]
            + w2 * g[2 * _NBOX:3 * _NBOX, :])

    lab_oh = (lax.broadcasted_iota(jnp.int32, (_NBOX, _NC), 1)
              == labc).astype(jnp.float32)

    sp_sum = jnp.zeros((_NBOX, 1), jnp.float32)
    xl_sum = jnp.zeros((_NBOX, 1), jnp.float32)
    box_num = jnp.float32(0.0)

    for s, (h, w) in enumerate(_SCALES):
        cx, cy, bw, bh, gi, gj = _box_geom(x1, y1, x2, y2, s)
        is_s = (scale_idx == s).astype(jnp.float32)
        for a in range(_NA):
            base = a * (5 + _NC)
            pxywh = gsel[:, base:base + 4]
            pxy = jax.nn.sigmoid(pxywh[:, 0:2])
            pwh = jax.nn.sigmoid(pxywh[:, 2:4])
            pcx = (gi.astype(jnp.float32) + pxy[:, 0:1]) / float(w)
            pcy = (gj.astype(jnp.float32) + pxy[:, 1:2]) / float(h)
            pw = pwh[:, 0:1]
            ph = pwh[:, 1:2]
            px1 = pcx - pw * 0.5
            py1 = pcy - ph * 0.5
            px2 = pcx + pw * 0.5
            py2 = pcy + ph * 0.5
            tx1 = cx - bw * 0.5
            ty1 = cy - bh * 0.5
            tx2 = cx + bw * 0.5
            ty2 = cy + bh * 0.5
            area1 = (px2 - px1) * (py2 - py1)
            area2 = (tx2 - tx1) * (ty2 - ty1)
            iw = jnp.maximum(jnp.minimum(px2, tx2) - jnp.maximum(px1, tx1),
                             0.0)
            ih = jnp.maximum(jnp.minimum(py2, ty2) - jnp.maximum(py1, ty1),
                             0.0)
            inter = iw * ih
            union = area1 + area2 - inter
            iou = inter / union
            cw = jnp.maximum(px2, tx2) - jnp.minimum(px1, tx1)
            chh = jnp.maximum(py2, ty2) - jnp.minimum(py1, ty1)
            areac = jnp.maximum(cw, 0.0) * jnp.maximum(chh, 0.0)
            giou = iou - (areac - union) / areac
            box_num = box_num + jnp.sum((1.0 - giou) * selv * is_s)

    for a in range(_NA):
        base = a * (5 + _NC)
        x_cls = gsel[:, base + 5:base + 5 + _NC]
        sp = jnp.maximum(x_cls, 0.0) + jnp.log1p(jnp.exp(-jnp.abs(x_cls)))
        sp_sum = sp_sum + jnp.sum(sp, axis=1, keepdims=True)
        xl_sum = xl_sum + jnp.sum(x_cls * lab_oh, axis=1, keepdims=True)

    cls_terms = []
    for s in range(3):
        is_s = (scale_idx == s).astype(jnp.float32)
        cls_terms.append(jnp.sum(sp_sum * rep_cell * is_s)
                         - jnp.sum(xl_sum * rep_cl * is_s))
    selsum3 = _NA * jnp.sum(selv)
    return cls_terms, box_num, selsum3


def _main_body(p3_ref, p4_ref, p5_ref, boxes_ref, labels_ref, g_ref,
               boxes2_ref, labels2_ref, boxest_ref, labelst_ref,
               out_ref, g_acc):
    b = pl.program_id(0)

    g_acc[pl.ds(b * (3 * _NBOX // _B), 3 * _NBOX // _B), :] = g_ref[...]

    boxes = boxes_ref[0]
    labels = labels_ref[0]
    lane = lax.broadcasted_iota(jnp.int32, (1, 128), 1)
    row = jnp.zeros((1, 128), jnp.float32)
    for s, pred_ref in enumerate((p3_ref, p4_ref, p5_ref)):
        pos_e, neg_e, cells = _obj_partials(pred_ref, boxes, labels, s)
        row = row + jnp.where(lane == 3 * s, pos_e, 0.0)
        row = row + jnp.where(lane == 3 * s + 1, neg_e, 0.0)
        row = row + jnp.where(lane == 3 * s + 2, cells, 0.0)

    @pl.when(b == 0)
    def _():
        out_ref[...] = jnp.zeros_like(out_ref)

    out_ref[...] += row

    @pl.when(b == _B - 1)
    def _():
        cls_terms, box_num, selsum3 = _sparse_terms(
            g_acc[...], boxes2_ref, labels2_ref, boxest_ref, labelst_ref)
        row2 = jnp.where(lane == 9, cls_terms[0], 0.0)
        row2 = row2 + jnp.where(lane == 10, cls_terms[1], 0.0)
        row2 = row2 + jnp.where(lane == 11, cls_terms[2], 0.0)
        row2 = row2 + jnp.where(lane == 12, box_num, 0.0)
        row2 = row2 + jnp.where(lane == 13, selsum3, 0.0)
        out_ref[...] += row2


def _run_main(p3_4, p4_4, p5_4, boxes, labels3, g2, boxes2, labels2,
              boxest, labelst):
    gs = 3 * _NBOX // _B
    return pl.pallas_call(
        _main_body,
        grid=(_B,),
        in_specs=[
            pl.BlockSpec((1, _NA, 8, 6400), lambda b: (b, 0, 0, 0)),
            pl.BlockSpec((1, _NA, 8, 1600), lambda b: (b, 0, 0, 0)),
            pl.BlockSpec((1, _NA, 8, 400), lambda b: (b, 0, 0, 0)),
            pl.BlockSpec((1, _NB, 4), lambda b: (b, 0, 0)),
            pl.BlockSpec((1, _NB, 1), lambda b: (b, 0, 0)),
            pl.BlockSpec((gs, 256), lambda b: (b, 0)),
            pl.BlockSpec((_NBOX, 4), lambda b: (0, 0)),
            pl.BlockSpec((_NBOX, 1), lambda b: (0, 0)),
            pl.BlockSpec((4, _NBOX), lambda b: (0, 0)),
            pl.BlockSpec((1, _NBOX), lambda b: (0, 0)),
        ],
        out_specs=pl.BlockSpec((1, 128), lambda b: (0, 0)),
        out_shape=jax.ShapeDtypeStruct((1, 128), jnp.float32),
        scratch_shapes=[pltpu.VMEM((3 * _NBOX, 256), jnp.float32)],
    )(p3_4, p4_4, p5_4, boxes, labels3, g2, boxes2, labels2, boxest, labelst)


def kernel(pred_p3, pred_p4, pred_p5, target_boxes, target_labels):
    boxes2 = target_boxes.reshape(_NBOX, 4)
    labels2 = target_labels.reshape(_NBOX, 1)
    boxest = jnp.transpose(boxes2)
    labelst = target_labels.reshape(1, _NBOX)
    labels3 = target_labels.reshape(_B, _NB, 1)

    idx = _prep(boxes2)
    idx2 = idx.reshape(3 * _NBOX * 2, 128)

    t3 = pred_p3.reshape(-1)
    t4 = pred_p4.reshape(-1)
    t5 = pred_p5.reshape(-1)
    g = _sc_gather(t3, t4, t5, idx2)
    g2 = g.reshape(3 * _NBOX, 256)

    p3_4 = pred_p3.reshape(_B, _NA, 5 + _NC, 6400)
    p4_4 = pred_p4.reshape(_B, _NA, 5 + _NC, 1600)
    p5_4 = pred_p5.reshape(_B, _NA, 5 + _NC, 400)
    r = _run_main(p3_4, p4_4, p5_4, target_boxes, labels3, g2,
                  boxes2, labels2, boxest, labelst)[0]

    obj = jnp.float32(0.0)
    cls = jnp.float32(0.0)
    for s in range(3):
        denom = jnp.maximum(r[3 * s + 2], 1.0)
        obj = obj + _L_OBJ * r[3 * s] / denom + _L_NOOBJ * r[3 * s + 1] / denom
        cls = cls + r[9 + s] / jnp.maximum(r[3 * s + 2] * _NC, 1.0)
    box = r[12] / jnp.maximum(r[13], 1.0)
    return obj + _L_CLS * cls + _L_BOX * box
